# Initial kernel scaffold; baseline (speedup 1.0000x reference)
import jax
import jax.numpy as jnp
from jax.experimental import pallas as pl


def kernel(x, quantiles):
    raise NotImplementedError("write your pallas kernel here")


# speedup vs baseline: 53.5801x; 53.5801x over previous
import functools

import jax
import jax.numpy as jnp
from jax import lax
from jax.experimental import pallas as pl
from jax.experimental.pallas import tpu as pltpu
from jax.experimental.pallas import tpu_sc as plsc

B = 4
N = 96 * 224 * 224
NC, NS = 2, 16
GROUP = 8
E = N // GROUP
CHUNK = 4096
NCHUNK = E // CHUNK
NBIN1 = 4096
NBIN2 = 256
NTGT = 2


def _sc_body(x_hbm, q_hbm, out_hbm, dbuf, hist1, hist2, red1,
             acc1, tmp1, red2, acc2, tmp2, qbuf, stage, sh1, sh2):
    c = lax.axis_index("c")
    s = lax.axis_index("s")
    sample = c * 2 + s // GROUP
    sj = s % GROUP
    base = sample * N + sj * E

    lane = lax.iota(jnp.int32, 16)
    ones16 = jnp.ones((16,), jnp.int32)
    zeros16 = jnp.zeros((16,), jnp.int32)
    lane_h1 = lane * NBIN1
    lane_h2 = lane * (NTGT * NBIN2)

    pltpu.sync_copy(q_hbm, qbuf)
    qv = qbuf[...]
    posv = qv * jnp.float32(N - 1) + jnp.float32(0.5)
    rkv = posv.astype(jnp.int32)
    rk0 = jnp.sum(jnp.where(lane == 0, rkv, 0))
    rk1 = jnp.sum(jnp.where(lane == 1, rkv, 0))

    def z1(i, _):
        hist1[pl.ds(i * 16, 16)] = zeros16
        return 0
    lax.fori_loop(0, NBIN1 * 16 // 16, z1, 0, unroll=8)

    def z2(i, _):
        hist2[pl.ds(i * 16, 16)] = zeros16
        return 0
    lax.fori_loop(0, NTGT * NBIN2 * 16 // 16, z2, 0, unroll=8)

    def monotone_key(v):
        u = plsc.bitcast(v, jnp.int32)
        m = lax.shift_right_arithmetic(u, 31) | jnp.int32(-(2 ** 31))
        return u ^ m

    def chunk1(ci, _):
        pltpu.sync_copy(x_hbm.at[pl.ds(base + ci * CHUNK, CHUNK)], dbuf)

        def vec1(vi, _):
            key = monotone_key(dbuf[pl.ds(vi * 16, 16)])
            bin1 = lax.shift_right_logical(key, 20)
            plsc.addupdate_scatter(hist1, [lane_h1 + bin1], ones16)
            return 0
        lax.fori_loop(0, CHUNK // 16, vec1, 0, unroll=8)
        return 0
    lax.fori_loop(0, NCHUNK, chunk1, 0)

    def r1(g, _):
        acc = zeros16
        for j in range(16):
            acc = acc + hist1[pl.ds(j * NBIN1 + g * 16, 16)]
        red1[pl.ds(g * 16, 16)] = acc
        return 0
    lax.fori_loop(0, NBIN1 // 16, r1, 0)
    pltpu.sync_copy(red1, sh1.at[s])
    plsc.subcore_barrier()

    g0 = (s // GROUP) * GROUP
    pltpu.sync_copy(sh1.at[g0], acc1)
    for j in range(1, GROUP):
        pltpu.sync_copy(sh1.at[g0 + j], tmp1)

        def addrow(g, _):
            acc1[pl.ds(g * 16, 16)] = (acc1[pl.ds(g * 16, 16)]
                                       + tmp1[pl.ds(g * 16, 16)])
            return 0
        lax.fori_loop(0, NBIN1 // 16, addrow, 0)

    def scan1(g, carry):
        run, bc0, bl0, bc1, bl1 = carry
        cum = plsc.cumsum(acc1[pl.ds(g * 16, 16)]) + run
        le0 = cum <= rk0
        le1 = cum <= rk1
        bc0 = bc0 + jnp.sum(jnp.where(le0, 1, 0))
        bl0 = jnp.maximum(bl0, jnp.max(jnp.where(le0, cum, 0)))
        bc1 = bc1 + jnp.sum(jnp.where(le1, 1, 0))
        bl1 = jnp.maximum(bl1, jnp.max(jnp.where(le1, cum, 0)))
        return (jnp.max(cum), bc0, bl0, bc1, bl1)

    z = jnp.int32(0)
    _, tb0, below0, tb1, below1 = lax.fori_loop(
        0, NBIN1 // 16, scan1, (z, z, z, z, z))
    rk0p = rk0 - below0
    rk1p = rk1 - below1

    def chunk2(ci, _):
        pltpu.sync_copy(x_hbm.at[pl.ds(base + ci * CHUNK, CHUNK)], dbuf)

        def vec2(vi, _):
            key = monotone_key(dbuf[pl.ds(vi * 16, 16)])
            bin1 = lax.shift_right_logical(key, 20)
            bin2 = lax.shift_right_logical(key, 12) & 0xFF
            idx2 = lane_h2 + bin2
            plsc.addupdate_scatter(hist2, [idx2], ones16, mask=bin1 == tb0)
            plsc.addupdate_scatter(hist2, [idx2 + NBIN2], ones16,
                                   mask=bin1 == tb1)
            return 0
        lax.fori_loop(0, CHUNK // 16, vec2, 0, unroll=8)
        return 0
    lax.fori_loop(0, NCHUNK, chunk2, 0)

    def r2(g, _):
        acc = zeros16
        for j in range(16):
            acc = acc + hist2[pl.ds(j * NTGT * NBIN2 + g * 16, 16)]
        red2[pl.ds(g * 16, 16)] = acc
        return 0
    lax.fori_loop(0, NTGT * NBIN2 // 16, r2, 0)
    pltpu.sync_copy(red2, sh2.at[s])
    plsc.subcore_barrier()

    @pl.when(sj == 0)
    def _():
        pltpu.sync_copy(sh2.at[g0], acc2)
        for j in range(1, GROUP):
            pltpu.sync_copy(sh2.at[g0 + j], tmp2)

            def addrow2(g, _):
                acc2[pl.ds(g * 16, 16)] = (acc2[pl.ds(g * 16, 16)]
                                           + tmp2[pl.ds(g * 16, 16)])
                return 0
            lax.fori_loop(0, NTGT * NBIN2 // 16, addrow2, 0)

        def scan2(g, carry):
            run0, bc0, run1, bc1 = carry
            c0 = plsc.cumsum(acc2[pl.ds(g * 16, 16)]) + run0
            c1 = plsc.cumsum(acc2[pl.ds(NTGT * NBIN2 // 2 + g * 16, 16)]) + run1
            bc0 = bc0 + jnp.sum(jnp.where(c0 <= rk0p, 1, 0))
            bc1 = bc1 + jnp.sum(jnp.where(c1 <= rk1p, 1, 0))
            return (jnp.max(c0), bc0, jnp.max(c1), bc1)

        _, sb0, _, sb1 = lax.fori_loop(0, NBIN2 // 16, scan2, (z, z, z, z))

        kv0 = lax.shift_left(tb0, 20) | lax.shift_left(sb0, 12) | 2048
        kv1 = lax.shift_left(tb1, 20) | lax.shift_left(sb1, 12) | 2048
        kvec = jnp.where(lane == 0, kv0, jnp.where(lane == 1, kv1, 0))
        top = lax.shift_right_logical(kvec, 31) == 1
        uvec = jnp.where(top, kvec ^ jnp.int32(-(2 ** 31)), ~kvec)
        fvec = plsc.bitcast(uvec, jnp.float32)
        vmin = jnp.sum(jnp.where(lane == 0, fvec, 0.0))
        vmax = jnp.sum(jnp.where(lane == 1, fvec, 0.0))
        denom = vmax - vmin
        safev = jnp.where(jnp.full((16,), denom) == 0.0,
                          jnp.full((16,), jnp.float32(1.0)),
                          jnp.full((16,), denom))
        invv = jnp.full((16,), jnp.float32(1.0)) / safev
        stage[...] = jnp.where(
            lane == 0, vmin,
            jnp.where(lane == 1, vmax,
                      jnp.where(lane == 2, invv, jnp.float32(0.0))))
        pltpu.sync_copy(stage, out_hbm.at[sample])


def _sc_quantiles(xflat, q16):
    mesh = plsc.VectorSubcoreMesh(core_axis_name="c", subcore_axis_name="s",
                                  num_cores=NC, num_subcores=NS)
    kern = pl.kernel(
        _sc_body,
        out_type=jax.ShapeDtypeStruct((B, 16), jnp.float32),
        mesh=mesh,
        scratch_types=[
            pltpu.VMEM((CHUNK,), jnp.float32),
            pltpu.VMEM((NBIN1 * 16,), jnp.int32),
            pltpu.VMEM((NTGT * NBIN2 * 16,), jnp.int32),
            pltpu.VMEM((NBIN1,), jnp.int32),
            pltpu.VMEM((NBIN1,), jnp.int32),
            pltpu.VMEM((NBIN1,), jnp.int32),
            pltpu.VMEM((NTGT * NBIN2,), jnp.int32),
            pltpu.VMEM((NTGT * NBIN2,), jnp.int32),
            pltpu.VMEM((NTGT * NBIN2,), jnp.int32),
            pltpu.VMEM((16,), jnp.float32),
            pltpu.VMEM((16,), jnp.float32),
            pltpu.VMEM_SHARED((NS, NBIN1), jnp.int32),
            pltpu.VMEM_SHARED((NS, NTGT * NBIN2), jnp.int32),
        ],
        compiler_params=pltpu.CompilerParams(needs_layout_passes=False),
    )
    return kern(xflat, q16)


def _tc_norm_body(params_ref, x_ref, o_ref):
    vmin = params_ref[0, 0, 0:1]
    vmax = params_ref[0, 0, 1:2]
    inv = params_ref[0, 0, 2:3]
    x = x_ref[...]
    o_ref[...] = (jnp.clip(x, vmin, vmax) - vmin) * inv


ROWS = 4704
COLS = 1024
RBLK = 672


def _tc_normalize(x3, params3):
    grid = (B, ROWS // RBLK)
    return pl.pallas_call(
        _tc_norm_body,
        grid=grid,
        in_specs=[
            pl.BlockSpec((1, 1, 128), lambda s_, c_: (s_, 0, 0)),
            pl.BlockSpec((1, RBLK, COLS), lambda s_, c_: (s_, c_, 0)),
        ],
        out_specs=pl.BlockSpec((1, RBLK, COLS), lambda s_, c_: (s_, c_, 0)),
        out_shape=jax.ShapeDtypeStruct((B, ROWS, COLS), jnp.float32),
    )(params3, x3)


def kernel(x, quantiles):
    xflat = x.reshape(-1)
    q16 = jnp.zeros((16,), jnp.float32).at[:2].set(quantiles)
    params = _sc_quantiles(xflat, q16)
    params3 = jnp.pad(params, ((0, 0), (0, 112))).reshape(B, 1, 128)
    x3 = x.reshape(B, ROWS, COLS)
    out3 = _tc_normalize(x3, params3)
    return out3.reshape(x.shape)


# speedup vs baseline: 61.8629x; 1.1546x over previous
import functools

import jax
import jax.numpy as jnp
from jax import lax
from jax.experimental import pallas as pl
from jax.experimental.pallas import tpu as pltpu
from jax.experimental.pallas import tpu_sc as plsc

B = 4
N = 96 * 224 * 224
NC, NS = 2, 16
GROUP = 8
E = N // GROUP
CHUNK = 6144
NCHUNK = E // CHUNK
NBIN1 = 4096
NBIN2 = 256
NTGT = 2


def _sc_body(x_hbm, q_hbm, out_hbm, dbuf, hist1, hist2, red1,
             acc1, tmp1, red2, acc2, tmp2, qbuf, stage, sem0, sem1,
             sh1, sh2):
    c = lax.axis_index("c")
    s = lax.axis_index("s")
    sample = c * 2 + s // GROUP
    sj = s % GROUP
    base = sample * N + sj * E

    lane = lax.iota(jnp.int32, 16)
    ones16 = jnp.ones((16,), jnp.int32)
    zeros16 = jnp.zeros((16,), jnp.int32)
    lane_h1 = lane * NBIN1
    lane_h2 = lane * (NTGT * NBIN2)

    pltpu.sync_copy(q_hbm, qbuf)
    qv = qbuf[...]
    posv = qv * jnp.float32(N - 1) + jnp.float32(0.5)
    rkv = posv.astype(jnp.int32)
    rk0 = jnp.sum(jnp.where(lane == 0, rkv, 0))
    rk1 = jnp.sum(jnp.where(lane == 1, rkv, 0))

    def z1(i, _):
        hist1[pl.ds(i * 16, 16)] = zeros16
        return 0
    lax.fori_loop(0, NBIN1 * 16 // 16, z1, 0, unroll=8)

    def z2(i, _):
        hist2[pl.ds(i * 16, 16)] = zeros16
        return 0
    lax.fori_loop(0, NTGT * NBIN2 * 16 // 16, z2, 0, unroll=8)

    def monotone_key(v):
        u = plsc.bitcast(v, jnp.int32)
        m = lax.shift_right_arithmetic(u, 31) | jnp.int32(-(2 ** 31))
        return u ^ m

    sems = (sem0, sem1)

    def stream(body_fn):
        for b in range(2):
            pltpu.async_copy(x_hbm.at[pl.ds(base + b * CHUNK, CHUNK)],
                             dbuf.at[b], sems[b])

        def ring(g, _):
            for b in range(2):
                pltpu.make_async_copy(
                    x_hbm.at[pl.ds(base, CHUNK)], dbuf.at[b], sems[b]).wait()
                body_fn(b)

                @pl.when(g + b + 2 < NCHUNK)
                def _():
                    pltpu.async_copy(
                        x_hbm.at[pl.ds(base + (g + b + 2) * CHUNK, CHUNK)],
                        dbuf.at[b], sems[b])
            return 0
        lax.fori_loop(0, NCHUNK // 2, lambda i, c: ring(i * 2, c), 0)

    def body1(slot):
        def vec1(vi, _):
            key = monotone_key(dbuf[slot, pl.ds(vi * 16, 16)])
            bin1 = lax.shift_right_logical(key, 20)
            plsc.addupdate_scatter(hist1, [lane_h1 + bin1], ones16)
            return 0
        lax.fori_loop(0, CHUNK // 16, vec1, 0, unroll=8)
    stream(body1)

    def r1(g, _):
        acc = zeros16
        for j in range(16):
            acc = acc + hist1[pl.ds(j * NBIN1 + g * 16, 16)]
        red1[pl.ds(g * 16, 16)] = acc
        return 0
    lax.fori_loop(0, NBIN1 // 16, r1, 0)
    pltpu.sync_copy(red1, sh1.at[s])
    plsc.subcore_barrier()

    g0 = (s // GROUP) * GROUP
    pltpu.sync_copy(sh1.at[g0], acc1)
    for j in range(1, GROUP):
        pltpu.sync_copy(sh1.at[g0 + j], tmp1)

        def addrow(g, _):
            acc1[pl.ds(g * 16, 16)] = (acc1[pl.ds(g * 16, 16)]
                                       + tmp1[pl.ds(g * 16, 16)])
            return 0
        lax.fori_loop(0, NBIN1 // 16, addrow, 0)

    def scan1(g, carry):
        run, bc0, bl0, bc1, bl1 = carry
        cum = plsc.cumsum(acc1[pl.ds(g * 16, 16)]) + run
        le0 = cum <= rk0
        le1 = cum <= rk1
        bc0 = bc0 + jnp.sum(jnp.where(le0, 1, 0))
        bl0 = jnp.maximum(bl0, jnp.max(jnp.where(le0, cum, 0)))
        bc1 = bc1 + jnp.sum(jnp.where(le1, 1, 0))
        bl1 = jnp.maximum(bl1, jnp.max(jnp.where(le1, cum, 0)))
        return (jnp.max(cum), bc0, bl0, bc1, bl1)

    z = jnp.int32(0)
    _, tb0, below0, tb1, below1 = lax.fori_loop(
        0, NBIN1 // 16, scan1, (z, z, z, z, z))
    rk0p = rk0 - below0
    rk1p = rk1 - below1

    def body2(slot):
        def vec2(vi, _):
            key = monotone_key(dbuf[slot, pl.ds(vi * 16, 16)])
            bin1 = lax.shift_right_logical(key, 20)
            bin2 = lax.shift_right_logical(key, 12) & 0xFF
            idx2 = lane_h2 + bin2
            plsc.addupdate_scatter(hist2, [idx2], ones16, mask=bin1 == tb0)
            plsc.addupdate_scatter(hist2, [idx2 + NBIN2], ones16,
                                   mask=bin1 == tb1)
            return 0
        lax.fori_loop(0, CHUNK // 16, vec2, 0, unroll=8)
    stream(body2)

    def r2(g, _):
        acc = zeros16
        for j in range(16):
            acc = acc + hist2[pl.ds(j * NTGT * NBIN2 + g * 16, 16)]
        red2[pl.ds(g * 16, 16)] = acc
        return 0
    lax.fori_loop(0, NTGT * NBIN2 // 16, r2, 0)
    pltpu.sync_copy(red2, sh2.at[s])
    plsc.subcore_barrier()

    @pl.when(sj == 0)
    def _():
        pltpu.sync_copy(sh2.at[g0], acc2)
        for j in range(1, GROUP):
            pltpu.sync_copy(sh2.at[g0 + j], tmp2)

            def addrow2(g, _):
                acc2[pl.ds(g * 16, 16)] = (acc2[pl.ds(g * 16, 16)]
                                           + tmp2[pl.ds(g * 16, 16)])
                return 0
            lax.fori_loop(0, NTGT * NBIN2 // 16, addrow2, 0)

        def scan2(g, carry):
            run0, bc0, run1, bc1 = carry
            c0 = plsc.cumsum(acc2[pl.ds(g * 16, 16)]) + run0
            c1 = plsc.cumsum(acc2[pl.ds(NTGT * NBIN2 // 2 + g * 16, 16)]) + run1
            bc0 = bc0 + jnp.sum(jnp.where(c0 <= rk0p, 1, 0))
            bc1 = bc1 + jnp.sum(jnp.where(c1 <= rk1p, 1, 0))
            return (jnp.max(c0), bc0, jnp.max(c1), bc1)

        _, sb0, _, sb1 = lax.fori_loop(0, NBIN2 // 16, scan2, (z, z, z, z))

        kv0 = lax.shift_left(tb0, 20) | lax.shift_left(sb0, 12) | 2048
        kv1 = lax.shift_left(tb1, 20) | lax.shift_left(sb1, 12) | 2048
        kvec = jnp.where(lane == 0, kv0, jnp.where(lane == 1, kv1, 0))
        top = lax.shift_right_logical(kvec, 31) == 1
        uvec = jnp.where(top, kvec ^ jnp.int32(-(2 ** 31)), ~kvec)
        fvec = plsc.bitcast(uvec, jnp.float32)
        vmin = jnp.sum(jnp.where(lane == 0, fvec, 0.0))
        vmax = jnp.sum(jnp.where(lane == 1, fvec, 0.0))
        denom = vmax - vmin
        safev = jnp.where(jnp.full((16,), denom) == 0.0,
                          jnp.full((16,), jnp.float32(1.0)),
                          jnp.full((16,), denom))
        invv = jnp.full((16,), jnp.float32(1.0)) / safev
        stage[...] = jnp.where(
            lane == 0, vmin,
            jnp.where(lane == 1, vmax,
                      jnp.where(lane == 2, invv, jnp.float32(0.0))))
        pltpu.sync_copy(stage, out_hbm.at[sample])


def _sc_quantiles(xflat, q16):
    mesh = plsc.VectorSubcoreMesh(core_axis_name="c", subcore_axis_name="s",
                                  num_cores=NC, num_subcores=NS)
    kern = pl.kernel(
        _sc_body,
        out_type=jax.ShapeDtypeStruct((B, 16), jnp.float32),
        mesh=mesh,
        scratch_types=[
            pltpu.VMEM((2, CHUNK), jnp.float32),
            pltpu.VMEM((NBIN1 * 16,), jnp.int32),
            pltpu.VMEM((NTGT * NBIN2 * 16,), jnp.int32),
            pltpu.VMEM((NBIN1,), jnp.int32),
            pltpu.VMEM((NBIN1,), jnp.int32),
            pltpu.VMEM((NBIN1,), jnp.int32),
            pltpu.VMEM((NTGT * NBIN2,), jnp.int32),
            pltpu.VMEM((NTGT * NBIN2,), jnp.int32),
            pltpu.VMEM((NTGT * NBIN2,), jnp.int32),
            pltpu.VMEM((16,), jnp.float32),
            pltpu.VMEM((16,), jnp.float32),
            pltpu.SemaphoreType.DMA,
            pltpu.SemaphoreType.DMA,
            pltpu.VMEM_SHARED((NS, NBIN1), jnp.int32),
            pltpu.VMEM_SHARED((NS, NTGT * NBIN2), jnp.int32),
        ],
        compiler_params=pltpu.CompilerParams(needs_layout_passes=False),
    )
    return kern(xflat, q16)


def _tc_norm_body(params_ref, x_ref, o_ref):
    vmin = params_ref[0, 0, 0:1]
    vmax = params_ref[0, 0, 1:2]
    inv = params_ref[0, 0, 2:3]
    x = x_ref[...]
    o_ref[...] = (jnp.clip(x, vmin, vmax) - vmin) * inv


ROWS = 4704
COLS = 1024
RBLK = 672


def _tc_normalize(x3, params3):
    grid = (B, ROWS // RBLK)
    return pl.pallas_call(
        _tc_norm_body,
        grid=grid,
        in_specs=[
            pl.BlockSpec((1, 1, 128), lambda s_, c_: (s_, 0, 0)),
            pl.BlockSpec((1, RBLK, COLS), lambda s_, c_: (s_, c_, 0)),
        ],
        out_specs=pl.BlockSpec((1, RBLK, COLS), lambda s_, c_: (s_, c_, 0)),
        out_shape=jax.ShapeDtypeStruct((B, ROWS, COLS), jnp.float32),
    )(params3, x3)


def kernel(x, quantiles):
    xflat = x.reshape(-1)
    q16 = jnp.zeros((16,), jnp.float32).at[:2].set(quantiles)
    params = _sc_quantiles(xflat, q16)
    params3 = jnp.pad(params, ((0, 0), (0, 112))).reshape(B, 1, 128)
    x3 = x.reshape(B, ROWS, COLS)
    out3 = _tc_normalize(x3, params3)
    return out3.reshape(x.shape)


# speedup vs baseline: 122.7352x; 1.9840x over previous
import functools

import jax
import jax.numpy as jnp
from jax import lax
from jax.experimental import pallas as pl
from jax.experimental.pallas import tpu as pltpu
from jax.experimental.pallas import tpu_sc as plsc

B = 4
N = 96 * 224 * 224
NC, NS = 2, 16
GROUP = 8
E = N // GROUP
CHUNK = 6144
NCHUNK = E // CHUNK
NBIN1 = 4096
NBIN2 = 256
NTGT = 2


def _sc_body(x_hbm, q_hbm, out_hbm, dbuf, hist1, hist2, red1,
             acc1, tmp1, red2, acc2, tmp2, qbuf, stage, sem0, sem1,
             sh1, sh2):
    c = lax.axis_index("c")
    s = lax.axis_index("s")
    sample = c * 2 + s // GROUP
    sj = s % GROUP
    base = sample * N + sj * E

    lane = lax.iota(jnp.int32, 16)
    ones16 = jnp.ones((16,), jnp.int32)
    zeros16 = jnp.zeros((16,), jnp.int32)
    lane_h1 = lane * NBIN1
    lane_h2 = lane * (NTGT * NBIN2)

    pltpu.sync_copy(q_hbm, qbuf)
    qv = qbuf[...]
    posv = qv * jnp.float32(N - 1) + jnp.float32(0.5)
    rkv = posv.astype(jnp.int32)
    rk0 = jnp.sum(jnp.where(lane == 0, rkv, 0))
    rk1 = jnp.sum(jnp.where(lane == 1, rkv, 0))

    def z1(i, _):
        hist1[pl.ds(i * 16, 16)] = zeros16
        return 0
    lax.fori_loop(0, NBIN1 * 16 // 16, z1, 0, unroll=8)

    def z2(i, _):
        hist2[pl.ds(i * 16, 16)] = zeros16
        return 0
    lax.fori_loop(0, NTGT * NBIN2 * 16 // 16, z2, 0, unroll=8)

    def monotone_key(v):
        u = plsc.bitcast(v, jnp.int32)
        m = lax.shift_right_arithmetic(u, 31) | jnp.int32(-(2 ** 31))
        return u ^ m

    sems = (sem0, sem1)

    def stream(body_fn):
        for b in range(2):
            pltpu.async_copy(x_hbm.at[pl.ds(base + b * CHUNK, CHUNK)],
                             dbuf.at[b], sems[b])

        def ring(g, _):
            for b in range(2):
                pltpu.make_async_copy(
                    x_hbm.at[pl.ds(base, CHUNK)], dbuf.at[b], sems[b]).wait()
                body_fn(b)

                @pl.when(g + b + 2 < NCHUNK)
                def _():
                    pltpu.async_copy(
                        x_hbm.at[pl.ds(base + (g + b + 2) * CHUNK, CHUNK)],
                        dbuf.at[b], sems[b])
            return 0
        lax.fori_loop(0, NCHUNK // 2, lambda i, c: ring(i * 2, c), 0)

    K = 8

    def body1(slot):
        def vec1(vi, _):
            vals = [dbuf[slot, pl.ds((vi * K + j) * 16, 16)] for j in range(K)]
            idxs = [lane_h1 + lax.shift_right_logical(monotone_key(v), 20)
                    for v in vals]
            for idx in idxs:
                plsc.addupdate_scatter(hist1, [idx], ones16)
            return 0
        lax.fori_loop(0, CHUNK // 16 // K, vec1, 0)
    stream(body1)

    def r1(g, _):
        acc = zeros16
        for j in range(16):
            acc = acc + hist1[pl.ds(j * NBIN1 + g * 16, 16)]
        red1[pl.ds(g * 16, 16)] = acc
        return 0
    lax.fori_loop(0, NBIN1 // 16, r1, 0)
    pltpu.sync_copy(red1, sh1.at[s])
    plsc.subcore_barrier()

    g0 = (s // GROUP) * GROUP
    pltpu.sync_copy(sh1.at[g0], acc1)
    for j in range(1, GROUP):
        pltpu.sync_copy(sh1.at[g0 + j], tmp1)

        def addrow(g, _):
            acc1[pl.ds(g * 16, 16)] = (acc1[pl.ds(g * 16, 16)]
                                       + tmp1[pl.ds(g * 16, 16)])
            return 0
        lax.fori_loop(0, NBIN1 // 16, addrow, 0)

    def scan1(g, carry):
        run, bc0, bl0, bc1, bl1 = carry
        cum = plsc.cumsum(acc1[pl.ds(g * 16, 16)]) + run
        le0 = cum <= rk0
        le1 = cum <= rk1
        bc0 = bc0 + jnp.sum(jnp.where(le0, 1, 0))
        bl0 = jnp.maximum(bl0, jnp.max(jnp.where(le0, cum, 0)))
        bc1 = bc1 + jnp.sum(jnp.where(le1, 1, 0))
        bl1 = jnp.maximum(bl1, jnp.max(jnp.where(le1, cum, 0)))
        return (jnp.max(cum), bc0, bl0, bc1, bl1)

    z = jnp.int32(0)
    _, tb0, below0, tb1, below1 = lax.fori_loop(
        0, NBIN1 // 16, scan1, (z, z, z, z, z))
    rk0p = rk0 - below0
    rk1p = rk1 - below1

    def body2(slot):
        def vec2(vi, _):
            vals = [dbuf[slot, pl.ds((vi * K + j) * 16, 16)] for j in range(K)]
            keys = [monotone_key(v) for v in vals]
            work = []
            for key in keys:
                bin1 = lax.shift_right_logical(key, 20)
                idx2 = lane_h2 + (lax.shift_right_logical(key, 12) & 0xFF)
                work.append((idx2, bin1 == tb0, bin1 == tb1))
            for idx2, m0, m1 in work:
                plsc.addupdate_scatter(hist2, [idx2], ones16, mask=m0)
                plsc.addupdate_scatter(hist2, [idx2 + NBIN2], ones16, mask=m1)
            return 0
        lax.fori_loop(0, CHUNK // 16 // K, vec2, 0)
    stream(body2)

    def r2(g, _):
        acc = zeros16
        for j in range(16):
            acc = acc + hist2[pl.ds(j * NTGT * NBIN2 + g * 16, 16)]
        red2[pl.ds(g * 16, 16)] = acc
        return 0
    lax.fori_loop(0, NTGT * NBIN2 // 16, r2, 0)
    pltpu.sync_copy(red2, sh2.at[s])
    plsc.subcore_barrier()

    @pl.when(sj == 0)
    def _():
        pltpu.sync_copy(sh2.at[g0], acc2)
        for j in range(1, GROUP):
            pltpu.sync_copy(sh2.at[g0 + j], tmp2)

            def addrow2(g, _):
                acc2[pl.ds(g * 16, 16)] = (acc2[pl.ds(g * 16, 16)]
                                           + tmp2[pl.ds(g * 16, 16)])
                return 0
            lax.fori_loop(0, NTGT * NBIN2 // 16, addrow2, 0)

        def scan2(g, carry):
            run0, bc0, run1, bc1 = carry
            c0 = plsc.cumsum(acc2[pl.ds(g * 16, 16)]) + run0
            c1 = plsc.cumsum(acc2[pl.ds(NTGT * NBIN2 // 2 + g * 16, 16)]) + run1
            bc0 = bc0 + jnp.sum(jnp.where(c0 <= rk0p, 1, 0))
            bc1 = bc1 + jnp.sum(jnp.where(c1 <= rk1p, 1, 0))
            return (jnp.max(c0), bc0, jnp.max(c1), bc1)

        _, sb0, _, sb1 = lax.fori_loop(0, NBIN2 // 16, scan2, (z, z, z, z))

        kv0 = lax.shift_left(tb0, 20) | lax.shift_left(sb0, 12) | 2048
        kv1 = lax.shift_left(tb1, 20) | lax.shift_left(sb1, 12) | 2048
        kvec = jnp.where(lane == 0, kv0, jnp.where(lane == 1, kv1, 0))
        top = lax.shift_right_logical(kvec, 31) == 1
        uvec = jnp.where(top, kvec ^ jnp.int32(-(2 ** 31)), ~kvec)
        fvec = plsc.bitcast(uvec, jnp.float32)
        vmin = jnp.sum(jnp.where(lane == 0, fvec, 0.0))
        vmax = jnp.sum(jnp.where(lane == 1, fvec, 0.0))
        denom = vmax - vmin
        safev = jnp.where(jnp.full((16,), denom) == 0.0,
                          jnp.full((16,), jnp.float32(1.0)),
                          jnp.full((16,), denom))
        invv = jnp.full((16,), jnp.float32(1.0)) / safev
        stage[...] = jnp.where(
            lane == 0, vmin,
            jnp.where(lane == 1, vmax,
                      jnp.where(lane == 2, invv, jnp.float32(0.0))))
        pltpu.sync_copy(stage, out_hbm.at[sample])


def _sc_quantiles(xflat, q16):
    mesh = plsc.VectorSubcoreMesh(core_axis_name="c", subcore_axis_name="s",
                                  num_cores=NC, num_subcores=NS)
    kern = pl.kernel(
        _sc_body,
        out_type=jax.ShapeDtypeStruct((B, 16), jnp.float32),
        mesh=mesh,
        scratch_types=[
            pltpu.VMEM((2, CHUNK), jnp.float32),
            pltpu.VMEM((NBIN1 * 16,), jnp.int32),
            pltpu.VMEM((NTGT * NBIN2 * 16,), jnp.int32),
            pltpu.VMEM((NBIN1,), jnp.int32),
            pltpu.VMEM((NBIN1,), jnp.int32),
            pltpu.VMEM((NBIN1,), jnp.int32),
            pltpu.VMEM((NTGT * NBIN2,), jnp.int32),
            pltpu.VMEM((NTGT * NBIN2,), jnp.int32),
            pltpu.VMEM((NTGT * NBIN2,), jnp.int32),
            pltpu.VMEM((16,), jnp.float32),
            pltpu.VMEM((16,), jnp.float32),
            pltpu.SemaphoreType.DMA,
            pltpu.SemaphoreType.DMA,
            pltpu.VMEM_SHARED((NS, NBIN1), jnp.int32),
            pltpu.VMEM_SHARED((NS, NTGT * NBIN2), jnp.int32),
        ],
        compiler_params=pltpu.CompilerParams(needs_layout_passes=False),
    )
    return kern(xflat, q16)


def _tc_norm_body(params_ref, x_ref, o_ref):
    vmin = params_ref[0, 0, 0:1]
    vmax = params_ref[0, 0, 1:2]
    inv = params_ref[0, 0, 2:3]
    x = x_ref[...]
    o_ref[...] = (jnp.clip(x, vmin, vmax) - vmin) * inv


ROWS = 4704
COLS = 1024
RBLK = 672


def _tc_normalize(x3, params3):
    grid = (B, ROWS // RBLK)
    return pl.pallas_call(
        _tc_norm_body,
        grid=grid,
        in_specs=[
            pl.BlockSpec((1, 1, 128), lambda s_, c_: (s_, 0, 0)),
            pl.BlockSpec((1, RBLK, COLS), lambda s_, c_: (s_, c_, 0)),
        ],
        out_specs=pl.BlockSpec((1, RBLK, COLS), lambda s_, c_: (s_, c_, 0)),
        out_shape=jax.ShapeDtypeStruct((B, ROWS, COLS), jnp.float32),
    )(params3, x3)


def kernel(x, quantiles):
    xflat = x.reshape(-1)
    q16 = jnp.zeros((16,), jnp.float32).at[:2].set(quantiles)
    params = _sc_quantiles(xflat, q16)
    params3 = jnp.pad(params, ((0, 0), (0, 112))).reshape(B, 1, 128)
    x3 = x.reshape(B, ROWS, COLS)
    out3 = _tc_normalize(x3, params3)
    return out3.reshape(x.shape)
